# baseline (speedup 1.0000x reference)
import functools

import jax
import jax.numpy as jnp
from jax import lax
from jax.experimental import pallas as pl
from jax.experimental.pallas import tpu as pltpu
from jax.experimental.pallas import tpu_sc as plsc

_NUM_CORES = 2
_NUM_SUBCORES = 16
_NUM_WORKERS = _NUM_CORES * _NUM_SUBCORES
_LANES = 16


def _matmul_body(emb_ref, w_ref, g_ref):
    g_ref[...] = lax.dot_general(
        emb_ref[...], w_ref[...],
        dimension_numbers=(((1,), (1,)), ((), ())),
        preferred_element_type=jnp.float32,
    )


def _logit_table(emb_table, out_weight):
    v = emb_table.shape[0]
    return pl.pallas_call(
        _matmul_body,
        out_shape=jax.ShapeDtypeStruct((v, v), jnp.float32),
    )(emb_table, out_weight)


@functools.partial(jax.jit, static_argnames=("batch", "seq", "vocab"))
def _sc_expand(g_flat, ids, batch, seq, vocab):
    mesh = plsc.VectorSubcoreMesh(
        core_axis_name="c", subcore_axis_name="s",
        num_cores=_NUM_CORES, num_subcores=_NUM_SUBCORES,
    )
    n_per_w = (batch * seq) // _NUM_WORKERS
    n_chunk = n_per_w // 4
    w_per_b = seq // n_per_w

    @functools.partial(
        pl.kernel,
        mesh=mesh,
        compiler_params=pltpu.CompilerParams(
            use_tc_tiling_on_sc=True, needs_layout_passes=False),
        out_type=jax.ShapeDtypeStruct((batch, seq, vocab), jnp.float32),
        scratch_types=[
            pltpu.VMEM((vocab, vocab), jnp.float32),
            pltpu.VMEM((n_per_w,), jnp.int32),
            pltpu.VMEM((2, n_chunk, vocab), jnp.float32),
            pltpu.SemaphoreType.DMA,
            pltpu.SemaphoreType.DMA,
            pltpu.SemaphoreType.DMA,
        ],
    )
    def expand_kernel(g_hbm, ids_hbm, out_hbm, g_v, idx_v, out_v,
                      sem_in, sem_a, sem_b):
        wid = lax.axis_index("s") * _NUM_CORES + lax.axis_index("c")
        bi = wid // w_per_b
        si = (wid % w_per_b) * n_per_w
        cg = pltpu.async_copy(g_hbm, g_v, sem_in)
        ci = pltpu.async_copy(ids_hbm.at[bi, pl.ds(si, n_per_w)], idx_v, sem_in)
        cg.wait()
        ci.wait()

        sems = (sem_a, sem_b)
        copies = [None, None]
        for c in range(n_per_w // n_chunk):
            cbase = c * n_chunk
            buf = c % 2
            if copies[buf] is not None:
                copies[buf].wait()

            @plsc.parallel_loop(0, n_chunk // _LANES)
            def block_body(b):
                base = b * _LANES
                idx = idx_v[pl.ds(cbase + base, _LANES)]
                for t in range(_LANES):
                    tok = base + t
                    goff = idx[t]
                    for h in range(vocab // _LANES):
                        out_v[buf, tok, pl.ds(h * _LANES, _LANES)] = (
                            g_v[goff, pl.ds(h * _LANES, _LANES)])

            copies[buf] = pltpu.async_copy(
                out_v.at[buf],
                out_hbm.at[bi, pl.ds(si + cbase, n_chunk)],
                sems[buf])
        for cp in copies:
            cp.wait()

    return expand_kernel(g_flat, ids)


def kernel(input_ids, emb_table, out_weight):
    batch, seq = input_ids.shape
    vocab = emb_table.shape[0]
    g = _logit_table(emb_table, out_weight)
    return _sc_expand(g, input_ids.astype(jnp.int32), batch, seq, vocab)


# speedup vs baseline: 1.0213x; 1.0213x over previous
import functools

import jax
import jax.numpy as jnp
from jax import lax
from jax.experimental import pallas as pl
from jax.experimental.pallas import tpu as pltpu
from jax.experimental.pallas import tpu_sc as plsc

_NUM_CORES = 2
_NUM_SUBCORES = 16
_NUM_WORKERS = _NUM_CORES * _NUM_SUBCORES
_LANES = 16


def _matmul_body(emb_ref, w_ref, g_ref):
    g_ref[...] = lax.dot_general(
        emb_ref[...], w_ref[...],
        dimension_numbers=(((1,), (1,)), ((), ())),
        preferred_element_type=jnp.float32,
    )


def _logit_table(emb_table, out_weight):
    v = emb_table.shape[0]
    return pl.pallas_call(
        _matmul_body,
        out_shape=jax.ShapeDtypeStruct((v, v), jnp.float32),
    )(emb_table, out_weight)


@functools.partial(jax.jit, static_argnames=("batch", "seq", "vocab"))
def _sc_expand(g_flat, ids, batch, seq, vocab):
    mesh = plsc.VectorSubcoreMesh(
        core_axis_name="c", subcore_axis_name="s",
        num_cores=_NUM_CORES, num_subcores=_NUM_SUBCORES,
    )
    n_per_w = (batch * seq) // _NUM_WORKERS
    n_chunk = n_per_w // 2
    w_per_b = seq // n_per_w

    @functools.partial(
        pl.kernel,
        mesh=mesh,
        compiler_params=pltpu.CompilerParams(
            use_tc_tiling_on_sc=True, needs_layout_passes=False),
        out_type=jax.ShapeDtypeStruct((batch, seq, vocab), jnp.float32),
        scratch_types=[
            pltpu.VMEM((vocab, vocab), jnp.float32),
            pltpu.VMEM((n_per_w,), jnp.int32),
            pltpu.VMEM((n_chunk, vocab), jnp.float32),
            pltpu.SemaphoreType.DMA,
        ],
    )
    def expand_kernel(g_hbm, ids_hbm, out_hbm, g_v, idx_v, out_v, sem_in):
        wid = lax.axis_index("s") * _NUM_CORES + lax.axis_index("c")
        bi = wid // w_per_b
        si = (wid % w_per_b) * n_per_w
        cg = pltpu.async_copy(g_hbm, g_v, sem_in)
        ci = pltpu.async_copy(ids_hbm.at[bi, pl.ds(si, n_per_w)], idx_v, sem_in)
        cg.wait()
        ci.wait()

        def chunk_body(c, _):
            cbase = c * n_chunk

            @plsc.parallel_loop(0, n_chunk // _LANES)
            def block_body(b):
                base = b * _LANES
                idx = idx_v[pl.ds(cbase + base, _LANES)]
                for t in range(_LANES):
                    tok = base + t
                    goff = idx[t]
                    for h in range(vocab // _LANES):
                        out_v[tok, pl.ds(h * _LANES, _LANES)] = (
                            g_v[goff, pl.ds(h * _LANES, _LANES)])

            pltpu.sync_copy(out_v, out_hbm.at[bi, pl.ds(si + cbase, n_chunk)])
            return _

        lax.fori_loop(0, n_per_w // n_chunk, chunk_body, 0)

    return expand_kernel(g_flat, ids)


def kernel(input_ids, emb_table, out_weight):
    batch, seq = input_ids.shape
    vocab = emb_table.shape[0]
    g = _logit_table(emb_table, out_weight)
    return _sc_expand(g, input_ids.astype(jnp.int32), batch, seq, vocab)


# speedup vs baseline: 1.0410x; 1.0193x over previous
import functools

import jax
import jax.numpy as jnp
from jax import lax
from jax.experimental import pallas as pl
from jax.experimental.pallas import tpu as pltpu
from jax.experimental.pallas import tpu_sc as plsc

_NUM_CORES = 2
_NUM_SUBCORES = 16
_NUM_WORKERS = _NUM_CORES * _NUM_SUBCORES
_LANES = 16


def _matmul_body(emb_ref, w_ref, g_ref):
    g_ref[...] = lax.dot_general(
        emb_ref[...], w_ref[...],
        dimension_numbers=(((1,), (1,)), ((), ())),
        preferred_element_type=jnp.float32,
    )


def _logit_table(emb_table, out_weight):
    v = emb_table.shape[0]
    return pl.pallas_call(
        _matmul_body,
        out_shape=jax.ShapeDtypeStruct((v, v), jnp.float32),
    )(emb_table, out_weight)


@functools.partial(jax.jit, static_argnames=("batch", "seq", "vocab"))
def _sc_expand(g_flat, ids, batch, seq, vocab):
    mesh = plsc.VectorSubcoreMesh(
        core_axis_name="c", subcore_axis_name="s",
        num_cores=_NUM_CORES, num_subcores=_NUM_SUBCORES,
    )
    n_per_w = (batch * seq) // _NUM_WORKERS
    n_chunk = n_per_w // 2
    w_per_b = seq // n_per_w

    @functools.partial(
        pl.kernel,
        mesh=mesh,
        compiler_params=pltpu.CompilerParams(
            use_tc_tiling_on_sc=True, needs_layout_passes=False),
        out_type=jax.ShapeDtypeStruct((batch, seq, vocab), jnp.float32),
        scratch_types=[
            pltpu.VMEM((vocab, vocab), jnp.float32),
            pltpu.VMEM((n_per_w,), jnp.int32),
            pltpu.VMEM((n_chunk, vocab), jnp.float32),
            pltpu.SemaphoreType.DMA,
        ],
    )
    def expand_kernel(g_hbm, ids_hbm, out_hbm, g_v, idx_v, out_v, sem_in):
        wid = lax.axis_index("s") * _NUM_CORES + lax.axis_index("c")
        bi = wid // w_per_b
        si = (wid % w_per_b) * n_per_w
        cg = pltpu.async_copy(g_hbm, g_v, sem_in)
        ci = pltpu.async_copy(ids_hbm.at[bi, pl.ds(si, n_per_w)], idx_v, sem_in)
        cg.wait()
        ci.wait()

        def chunk_body(c, _):
            cbase = c * n_chunk

            @plsc.parallel_loop(0, n_chunk // _LANES, unroll=2)
            def block_body(b):
                base = b * _LANES
                idx = idx_v[pl.ds(cbase + base, _LANES)]
                for t in range(_LANES):
                    tok = base + t
                    goff = idx[t]
                    for h in range(vocab // _LANES):
                        out_v[tok, pl.ds(h * _LANES, _LANES)] = (
                            g_v[goff, pl.ds(h * _LANES, _LANES)])

            pltpu.sync_copy(out_v, out_hbm.at[bi, pl.ds(si + cbase, n_chunk)])
            return _

        lax.fori_loop(0, n_per_w // n_chunk, chunk_body, 0)

    return expand_kernel(g_flat, ids)


def kernel(input_ids, emb_table, out_weight):
    batch, seq = input_ids.shape
    vocab = emb_table.shape[0]
    g = _logit_table(emb_table, out_weight)
    return _sc_expand(g, input_ids.astype(jnp.int32), batch, seq, vocab)


# speedup vs baseline: 1.0436x; 1.0025x over previous
import functools

import jax
import jax.numpy as jnp
from jax import lax
from jax.experimental import pallas as pl
from jax.experimental.pallas import tpu as pltpu
from jax.experimental.pallas import tpu_sc as plsc

_NUM_CORES = 2
_NUM_SUBCORES = 16
_NUM_WORKERS = _NUM_CORES * _NUM_SUBCORES
_LANES = 16


def _matmul_body(emb_ref, w_ref, g_ref):
    g_ref[...] = lax.dot_general(
        emb_ref[...], w_ref[...],
        dimension_numbers=(((1,), (1,)), ((), ())),
        preferred_element_type=jnp.float32,
    )


def _logit_table(emb_table, out_weight):
    v = emb_table.shape[0]
    return pl.pallas_call(
        _matmul_body,
        out_shape=jax.ShapeDtypeStruct((v, v), jnp.float32),
    )(emb_table, out_weight)


@functools.partial(jax.jit, static_argnames=("batch", "seq", "vocab"))
def _sc_expand(g_flat, ids, batch, seq, vocab):
    mesh = plsc.VectorSubcoreMesh(
        core_axis_name="c", subcore_axis_name="s",
        num_cores=_NUM_CORES, num_subcores=_NUM_SUBCORES,
    )
    n_per_w = (batch * seq) // _NUM_WORKERS
    n_chunk = n_per_w // 2
    w_per_b = seq // n_per_w

    @functools.partial(
        pl.kernel,
        mesh=mesh,
        compiler_params=pltpu.CompilerParams(
            use_tc_tiling_on_sc=True, needs_layout_passes=False),
        out_type=jax.ShapeDtypeStruct((batch, seq, vocab), jnp.float32),
        scratch_types=[
            pltpu.VMEM((vocab, vocab), jnp.float32),
            pltpu.VMEM((n_per_w,), jnp.int32),
            pltpu.VMEM((n_chunk, vocab), jnp.float32),
            pltpu.SemaphoreType.DMA,
        ],
    )
    def expand_kernel(g_hbm, ids_hbm, out_hbm, g_v, idx_v, out_v, sem_in):
        wid = lax.axis_index("s") * _NUM_CORES + lax.axis_index("c")
        bi = wid // w_per_b
        si = (wid % w_per_b) * n_per_w
        cg = pltpu.async_copy(g_hbm, g_v, sem_in)
        ci = pltpu.async_copy(ids_hbm.at[bi, pl.ds(si, n_per_w)], idx_v, sem_in)
        cg.wait()
        ci.wait()

        def chunk_body(c, _):
            cbase = c * n_chunk

            @plsc.parallel_loop(0, n_chunk // _LANES, unroll=4)
            def block_body(b):
                base = b * _LANES
                idx = idx_v[pl.ds(cbase + base, _LANES)]
                for t in range(_LANES):
                    tok = base + t
                    goff = idx[t]
                    for h in range(vocab // _LANES):
                        out_v[tok, pl.ds(h * _LANES, _LANES)] = (
                            g_v[goff, pl.ds(h * _LANES, _LANES)])

            pltpu.sync_copy(out_v, out_hbm.at[bi, pl.ds(si + cbase, n_chunk)])
            return _

        lax.fori_loop(0, n_per_w // n_chunk, chunk_body, 0)

    return expand_kernel(g_flat, ids)


def kernel(input_ids, emb_table, out_weight):
    batch, seq = input_ids.shape
    vocab = emb_table.shape[0]
    g = _logit_table(emb_table, out_weight)
    return _sc_expand(g, input_ids.astype(jnp.int32), batch, seq, vocab)
